# baseline (speedup 1.0000x reference)
import math

import numpy as np

import jax
import jax.numpy as jnp
from jax.experimental import pallas as pl
from jax.experimental.pallas import tpu as pltpu

_L = 2048
_D = 64
_H = 16
_SK = 40
_NTOP = 40
_TJ = 256


def _threefry2x32(k1, k2, x0, x1):
    ks = [np.uint32(k1), np.uint32(k2),
          np.uint32(np.uint32(k1) ^ np.uint32(k2) ^ np.uint32(0x1BD11BDA))]
    x0 = (x0 + ks[0]).astype(np.uint32)
    x1 = (x1 + ks[1]).astype(np.uint32)
    rots = [(13, 15, 26, 6), (17, 29, 16, 24)]
    adds = [(1, 2, 1), (2, 0, 2), (0, 1, 3), (1, 2, 4), (2, 0, 5)]
    for g in range(5):
        for r in rots[g % 2]:
            r = np.uint32(r)
            x0 = (x0 + x1).astype(np.uint32)
            x1 = ((x1 << r) | (x1 >> np.uint32(32 - r))).astype(np.uint32)
            x1 = x0 ^ x1
        a, b, inc = adds[g]
        x0 = (x0 + ks[a]).astype(np.uint32)
        x1 = (x1 + ks[b] + np.uint32(inc)).astype(np.uint32)
    return x0, x1


def _build_cnt_t() -> np.ndarray:
    b1, b2 = _threefry2x32(0, 42, np.zeros(2, np.uint32),
                           np.arange(2, dtype=np.uint32))
    n = _L * _SK
    o1, o2 = _threefry2x32(b1[1], b2[1], np.zeros(n, np.uint32),
                           np.arange(n, dtype=np.uint32))
    idx = ((o1 ^ o2) % np.uint32(_L)).astype(np.int64)
    cnt_t = np.zeros((_L, _L), np.float32)
    np.add.at(cnt_t, (idx, np.repeat(np.arange(_L), _SK)), 1.0)
    return cnt_t


_CNT_T = _build_cnt_t()
_MASKADD_T = np.where(_CNT_T > 0.0, 0.0, -1e30).astype(np.float32)
_CNT_N = np.ascontiguousarray(_CNT_T.T)


def _prob_attn_kernel(q_ref, k_ref, v_ref, cnt_ref, ma_ref, o_ref,
                      m_ref, idx_ref):
    p = pl.program_id(0)
    h = pl.program_id(1)
    li = jax.lax.broadcasted_iota(jnp.int32, (1, _L), 1)

    @pl.when(p == 0)
    def _stage1():
        for sub in range(2):
            q = q_ref[:, sub * _D:(sub + 1) * _D]
            k = k_ref[:, sub * _D:(sub + 1) * _D]
            mx = jnp.full((1, _L), -jnp.inf, jnp.float32)
            for t in range(_L // _TJ):
                kt = k[t * _TJ:(t + 1) * _TJ, :]
                st = jax.lax.dot_general(
                    kt, q, (((1,), (1,)), ((), ())),
                    preferred_element_type=jnp.float32)
                mx = jnp.maximum(
                    mx,
                    jnp.max(st + ma_ref[t * _TJ:(t + 1) * _TJ, :], axis=0,
                            keepdims=True))
            ksum = jnp.dot(cnt_ref[...], k,
                           preferred_element_type=jnp.float32)
            sm = jnp.sum(q * ksum, axis=1).reshape(1, _L)
            m_ref[pl.ds(h * 2 + sub, 1), :] = mx - sm * (1.0 / _L)

    @pl.when((p == 1) & (h == 0))
    def _topk():
        lane = jax.lax.broadcasted_iota(jnp.int32, (_H, _L), 1)
        lane_u = jax.lax.broadcasted_iota(jnp.int32, (_H, 128), 1)

        def topk_body(u, carry):
            rmax = jnp.max(carry, axis=1, keepdims=True)
            i = jnp.min(jnp.where(carry == rmax, lane, _L),
                        axis=1, keepdims=True)
            idx_ref[...] = jnp.where(lane_u == u, i, idx_ref[...])
            return jnp.where(lane == i, -jnp.inf, carry)

        jax.lax.fori_loop(0, _NTOP, topk_body, m_ref[...])

    @pl.when(p == 1)
    def _stage2():
        for sub in range(2):
            q = q_ref[:, sub * _D:(sub + 1) * _D]
            k = k_ref[:, sub * _D:(sub + 1) * _D]
            v = v_ref[:, sub * _D:(sub + 1) * _D]
            idxh = idx_ref[pl.ds(h * 2 + sub, 1), :_NTOP].reshape(_NTOP, 1)
            oh = (idxh == li).astype(jnp.float32)
            qr = jnp.dot(oh, q, preferred_element_type=jnp.float32)
            s2 = jax.lax.dot_general(
                qr, k, (((1,), (1,)), ((), ())),
                preferred_element_type=jnp.float32)
            s2 = s2 * (1.0 / math.sqrt(_D))
            s2 = s2 - jnp.max(s2, axis=1, keepdims=True)
            pr = jnp.exp(s2)
            pr = pr / jnp.sum(pr, axis=1, keepdims=True)
            upd = jnp.dot(pr, v, preferred_element_type=jnp.float32)
            vmean = jnp.mean(v, axis=0, keepdims=True)
            scat = jax.lax.dot_general(
                oh, upd - vmean, (((0,), (0,)), ((), ())),
                preferred_element_type=jnp.float32)
            o_ref[:, sub * _D:(sub + 1) * _D] = scat + vmean


def kernel(queries, keys, values):
    b, l, h, d = queries.shape
    q2 = queries.reshape(l, h * d)
    k2 = keys.reshape(l, h * d)
    v2 = values.reshape(l, h * d)
    out = pl.pallas_call(
        _prob_attn_kernel,
        grid=(2, h // 2),
        in_specs=[
            pl.BlockSpec((l, 2 * d), lambda p, i: (0, i)),
            pl.BlockSpec((l, 2 * d), lambda p, i: (0, i)),
            pl.BlockSpec((l, 2 * d), lambda p, i: (0, i)),
            pl.BlockSpec((_L, _L), lambda p, i: (0, 0)),
            pl.BlockSpec((_L, _L), lambda p, i: (0, 0)),
        ],
        out_specs=pl.BlockSpec((l, 2 * d), lambda p, i: (0, i * p)),
        out_shape=jax.ShapeDtypeStruct((l, h * d), jnp.float32),
        scratch_shapes=[
            pltpu.VMEM((_H, _L), jnp.float32),
            pltpu.VMEM((_H, 128), jnp.int32),
        ],
    )(q2, k2, v2, _CNT_N, _MASKADD_T)
    return out.reshape(b, l, h, d)


# speedup vs baseline: 1.4523x; 1.4523x over previous
import math

import numpy as np

import jax
import jax.numpy as jnp
from jax.experimental import pallas as pl
from jax.experimental.pallas import tpu as pltpu

_L = 2048
_D = 64
_H = 16
_SK = 40
_NTOP = 40
_TJ = 512


def _threefry2x32(k1, k2, x0, x1):
    ks = [np.uint32(k1), np.uint32(k2),
          np.uint32(np.uint32(k1) ^ np.uint32(k2) ^ np.uint32(0x1BD11BDA))]
    x0 = (x0 + ks[0]).astype(np.uint32)
    x1 = (x1 + ks[1]).astype(np.uint32)
    rots = [(13, 15, 26, 6), (17, 29, 16, 24)]
    adds = [(1, 2, 1), (2, 0, 2), (0, 1, 3), (1, 2, 4), (2, 0, 5)]
    for g in range(5):
        for r in rots[g % 2]:
            r = np.uint32(r)
            x0 = (x0 + x1).astype(np.uint32)
            x1 = ((x1 << r) | (x1 >> np.uint32(32 - r))).astype(np.uint32)
            x1 = x0 ^ x1
        a, b, inc = adds[g]
        x0 = (x0 + ks[a]).astype(np.uint32)
        x1 = (x1 + ks[b] + np.uint32(inc)).astype(np.uint32)
    return x0, x1


def _build_cnt_t() -> np.ndarray:
    b1, b2 = _threefry2x32(0, 42, np.zeros(2, np.uint32),
                           np.arange(2, dtype=np.uint32))
    n = _L * _SK
    o1, o2 = _threefry2x32(b1[1], b2[1], np.zeros(n, np.uint32),
                           np.arange(n, dtype=np.uint32))
    idx = ((o1 ^ o2) % np.uint32(_L)).astype(np.int64)
    cnt_t = np.zeros((_L, _L), np.float32)
    np.add.at(cnt_t, (idx, np.repeat(np.arange(_L), _SK)), 1.0)
    return cnt_t


_CNT_T = _build_cnt_t()
_MASKADD_T = np.where(_CNT_T > 0.0, 0.0, -1e30).astype(np.float32)


def _prob_attn_kernel(q_ref, k_ref, v_ref, cnt_ref, ma_ref, o_ref,
                      m_ref, idx_ref):
    p = pl.program_id(0)
    h = pl.program_id(1)
    li = jax.lax.broadcasted_iota(jnp.int32, (1, _L), 1)

    @pl.when(p == 0)
    def _stage1():
        for sub in range(2):
            q = q_ref[:, sub * _D:(sub + 1) * _D]
            k = k_ref[:, sub * _D:(sub + 1) * _D]
            mx = jnp.full((1, _L), -jnp.inf, jnp.float32)
            sm = jnp.zeros((1, _L), jnp.float32)
            for t in range(_L // _TJ):
                kt = k[t * _TJ:(t + 1) * _TJ, :]
                st = jax.lax.dot_general(
                    kt, q, (((1,), (1,)), ((), ())),
                    preferred_element_type=jnp.float32)
                mx = jnp.maximum(
                    mx,
                    jnp.max(st + ma_ref[t * _TJ:(t + 1) * _TJ, :], axis=0,
                            keepdims=True))
                sm = sm + jnp.sum(st * cnt_ref[t * _TJ:(t + 1) * _TJ, :],
                                  axis=0, keepdims=True)
            m_ref[pl.ds(h * 2 + sub, 1), :] = mx - sm * (1.0 / _L)

    @pl.when((p == 1) & (h == 0))
    def _topk():
        lane = jax.lax.broadcasted_iota(jnp.int32, (_H, _L), 1)
        lane_u = jax.lax.broadcasted_iota(jnp.int32, (_H, 128), 1)

        def topk_body(u, carry):
            rmax = jnp.max(carry, axis=1, keepdims=True)
            i = jnp.min(jnp.where(carry == rmax, lane, _L),
                        axis=1, keepdims=True)
            idx_ref[...] = jnp.where(lane_u == u, i, idx_ref[...])
            return jnp.where(lane == i, -jnp.inf, carry)

        jax.lax.fori_loop(0, _NTOP, topk_body, m_ref[...])

    @pl.when(p == 1)
    def _stage2():
        for sub in range(2):
            q = q_ref[:, sub * _D:(sub + 1) * _D]
            k = k_ref[:, sub * _D:(sub + 1) * _D]
            v = v_ref[:, sub * _D:(sub + 1) * _D]
            idxh = idx_ref[pl.ds(h * 2 + sub, 1), :_NTOP].reshape(_NTOP, 1)
            oh = (idxh == li).astype(jnp.float32)
            qr = jnp.dot(oh, q, preferred_element_type=jnp.float32)
            s2 = jax.lax.dot_general(
                qr, k, (((1,), (1,)), ((), ())),
                preferred_element_type=jnp.float32)
            s2 = s2 * (1.0 / math.sqrt(_D))
            s2 = s2 - jnp.max(s2, axis=1, keepdims=True)
            pr = jnp.exp(s2)
            pr = pr / jnp.sum(pr, axis=1, keepdims=True)
            upd = jnp.dot(pr, v, preferred_element_type=jnp.float32)
            vmean = jnp.mean(v, axis=0, keepdims=True)
            scat = jax.lax.dot_general(
                oh, upd - vmean, (((0,), (0,)), ((), ())),
                preferred_element_type=jnp.float32)
            o_ref[:, sub * _D:(sub + 1) * _D] = scat + vmean


def kernel(queries, keys, values):
    b, l, h, d = queries.shape
    q2 = queries.reshape(l, h * d)
    k2 = keys.reshape(l, h * d)
    v2 = values.reshape(l, h * d)
    out = pl.pallas_call(
        _prob_attn_kernel,
        grid=(2, h // 2),
        in_specs=[
            pl.BlockSpec((l, 2 * d), lambda p, i: (0, i)),
            pl.BlockSpec((l, 2 * d), lambda p, i: (0, i)),
            pl.BlockSpec((l, 2 * d), lambda p, i: (0, i)),
            pl.BlockSpec((_L, _L), lambda p, i: (0, 0)),
            pl.BlockSpec((_L, _L), lambda p, i: (0, 0)),
        ],
        out_specs=pl.BlockSpec((l, 2 * d), lambda p, i: (0, i * p)),
        out_shape=jax.ShapeDtypeStruct((l, h * d), jnp.float32),
        scratch_shapes=[
            pltpu.VMEM((_H, _L), jnp.float32),
            pltpu.VMEM((_H, 128), jnp.int32),
        ],
    )(q2, k2, v2, _CNT_T, _MASKADD_T)
    return out.reshape(b, l, h, d)
